# Initial kernel scaffold; baseline (speedup 1.0000x reference)
import jax
import jax.numpy as jnp
from jax.experimental import pallas as pl


def kernel(x, edge_index, edge_attr, mW1, mb1, mW2, mb2, mW3, mb3, uW1, ub1, uW2, ub2, uW3, ub3):
    raise NotImplementedError("write your pallas kernel here")


# speedup vs baseline: 17.7871x; 17.7871x over previous
import functools

import jax
import jax.numpy as jnp
from jax import lax
from jax.experimental import pallas as pl
from jax.experimental.pallas import tpu as pltpu
from jax.experimental.pallas import tpu_sc as plsc

N = 50000
E = 1600000
G = 50
B = 128
NUM_ITER = 3

NC = 2
NS = 16
NW = NC * NS
EPW = E // NW
CH = 2000
N_PAD = 51200

TE = 6400
TN = 6400

@functools.lru_cache(maxsize=None)
def _sc_kernels():
    mesh = plsc.VectorSubcoreMesh(
        core_axis_name="c", subcore_axis_name="s",
        num_cores=NC, num_subcores=NS)

    def wid():
        return lax.axis_index("s") * NC + lax.axis_index("c")

    sc_params = pltpu.CompilerParams(
        needs_layout_passes=False, use_tc_tiling_on_sc=False)

    @functools.partial(
        pl.kernel,
        out_type=jax.ShapeDtypeStruct((E,), jnp.float32),
        mesh=mesh,
        scratch_types=[
            pltpu.VMEM((N,), jnp.float32),
            pltpu.VMEM((CH,), jnp.int32),
            pltpu.VMEM((CH,), jnp.float32),
        ],
        compiler_params=sc_params,
    )
    def sc_gather(h_hbm, src_hbm, out_hbm, h_v, idx_v, xj_v):
        w = wid()
        pltpu.sync_copy(h_hbm, h_v)

        def chunk_body(ci, _):
            base = w * EPW + ci * CH
            pltpu.sync_copy(src_hbm.at[pl.ds(base, CH)], idx_v)

            def vec_body(j, _):
                idx = idx_v[pl.ds(j * 16, 16)]
                xj_v[pl.ds(j * 16, 16)] = plsc.load_gather(h_v, [idx])
                return 0

            lax.fori_loop(0, CH // 16, vec_body, 0)
            pltpu.sync_copy(xj_v, out_hbm.at[pl.ds(base, CH)])
            return 0

        lax.fori_loop(0, EPW // CH, chunk_body, 0)

    @functools.partial(
        pl.kernel,
        out_type=jax.ShapeDtypeStruct((NW, N_PAD), jnp.float32),
        mesh=mesh,
        scratch_types=[
            pltpu.VMEM((N_PAD,), jnp.float32),
            pltpu.VMEM((CH,), jnp.int32),
            pltpu.VMEM((CH,), jnp.float32),
        ],
        compiler_params=sc_params,
    )
    def sc_scatter_min(msg_hbm, dst_hbm, part_hbm, acc_v, idx_v, msg_v):
        w = wid()
        inf16 = jnp.full((16,), jnp.inf, dtype=jnp.float32)

        def init_body(i, _):
            acc_v[pl.ds(i * 16, 16)] = inf16
            return 0

        lax.fori_loop(0, N_PAD // 16, init_body, 0)

        def chunk_body(ci, _):
            base = w * EPW + ci * CH
            pltpu.sync_copy(dst_hbm.at[pl.ds(base, CH)], idx_v)
            pltpu.sync_copy(msg_hbm.at[pl.ds(base, CH)], msg_v)

            def vec_body(j, _):
                idx = idx_v[pl.ds(j * 16, 16)]
                m = msg_v[pl.ds(j * 16, 16)]
                cur = plsc.load_gather(acc_v, [idx])
                plsc.store_scatter(acc_v, [idx], jnp.minimum(cur, m))
                chk = plsc.load_gather(acc_v, [idx])

                def fix_body(pend):
                    plsc.store_scatter(acc_v, [idx], m, mask=pend)
                    c2 = plsc.load_gather(acc_v, [idx])
                    return c2 > m

                lax.while_loop(lambda p: jnp.any(p), fix_body, chk > m)
                return 0

            lax.fori_loop(0, CH // 16, vec_body, 0)
            return 0

        lax.fori_loop(0, EPW // CH, chunk_body, 0)
        pltpu.sync_copy(acc_v, part_hbm.at[wid()])

    return sc_gather, sc_scatter_min


def _bdot(a, b):
    return jax.lax.dot(a.astype(jnp.bfloat16), b.astype(jnp.bfloat16),
                       preferred_element_type=jnp.float32)


def _msg_mlp_body(xj_ref, ea_ref, w1t_ref, b1_ref, w2t_ref, b2_ref,
                  w3t_ref, b3_ref, out_ref):
    minp = jnp.concatenate([xj_ref[0], ea_ref[0]], axis=0)
    h1 = jnp.maximum(_bdot(w1t_ref[...], minp) + b1_ref[...], 0.0)
    h2 = jnp.maximum(_bdot(w2t_ref[...], h1) + b2_ref[...], 0.0)
    out_ref[0] = _bdot(w3t_ref[...], h2) + b3_ref[...]


def _tc_msg(xj, ea, w1t, b1c, w2t, b2c, w3t, b3c):
    nblk = E // TE
    xj3 = xj.reshape(nblk, 1, TE)
    ea3 = ea.reshape(nblk, 1, TE)
    full = lambda *s: pl.BlockSpec(s, lambda i: (0,) * len(s))
    out = pl.pallas_call(
        _msg_mlp_body,
        grid=(nblk,),
        in_specs=[
            pl.BlockSpec((1, 1, TE), lambda i: (i, 0, 0)),
            pl.BlockSpec((1, 1, TE), lambda i: (i, 0, 0)),
            full(B, 2), full(B, 1), full(B, B), full(B, 1),
            full(1, B), full(1, 1),
        ],
        out_specs=pl.BlockSpec((1, 1, TE), lambda i: (i, 0, 0)),
        out_shape=jax.ShapeDtypeStruct((nblk, 1, TE), jnp.float32),
    )(xj3, ea3, w1t, b1c, w2t, b2c, w3t, b3c)
    return out.reshape(E)


def _update_body(part_ref, w1c_ref, b1_ref, w2t_ref, b2_ref,
                 w3t_ref, b3_ref, out_ref):
    agg = jnp.min(part_ref[...], axis=0, keepdims=True)
    agg = jnp.where(jnp.isfinite(agg), agg, 0.0)
    h1 = jnp.maximum(_bdot(w1c_ref[...], agg) + b1_ref[...], 0.0)
    h2 = jnp.maximum(_bdot(w2t_ref[...], h1) + b2_ref[...], 0.0)
    out_ref[0] = _bdot(w3t_ref[...], h2) + b3_ref[...]


def _tc_update(part, w1c, b1c, w2t, b2c, w3t, b3c):
    nblk = N_PAD // TN
    full = lambda *s: pl.BlockSpec(s, lambda i: (0,) * len(s))
    out = pl.pallas_call(
        _update_body,
        grid=(nblk,),
        in_specs=[
            pl.BlockSpec((NW, TN), lambda i: (0, i)),
            full(B, 1), full(B, 1), full(B, B), full(B, 1),
            full(1, B), full(1, 1),
        ],
        out_specs=pl.BlockSpec((1, 1, TN), lambda i: (i, 0, 0)),
        out_shape=jax.ShapeDtypeStruct((nblk, 1, TN), jnp.float32),
    )(part, w1c, b1c, w2t, b2c, w3t, b3c)
    return out.reshape(N_PAD)[:N]


def kernel(x, edge_index, edge_attr, mW1, mb1, mW2, mb2, mW3, mb3,
           uW1, ub1, uW2, ub2, uW3, ub3):
    src = edge_index[0]
    dst = edge_index[1]

    mw1t = mW1.T
    mb1c = mb1.reshape(B, 1)
    mw2t = mW2.T
    mb2c = mb2.reshape(B, 1)
    mw3t = mW3.T
    mb3c = mb3.reshape(1, 1)
    uw1c = uW1.reshape(B, 1)
    ub1c = ub1.reshape(B, 1)
    uw2t = uW2.T
    ub2c = ub2.reshape(B, 1)
    uw3t = uW3.T
    ub3c = ub3.reshape(1, 1)

    sc_gather, sc_scatter_min = _sc_kernels()
    h = x.reshape(N)
    for _ in range(NUM_ITER):
        xj = sc_gather(h, src)
        msg = _tc_msg(xj, edge_attr, mw1t, mb1c, mw2t, mb2c, mw3t, mb3c)
        part = sc_scatter_min(msg, dst)
        h = _tc_update(part, uw1c, ub1c, uw2t, ub2c, uw3t, ub3c)
    return h.reshape(G, N // G)


# speedup vs baseline: 21.1814x; 1.1908x over previous
import functools

import jax
import jax.numpy as jnp
from jax import lax
from jax.experimental import pallas as pl
from jax.experimental.pallas import tpu as pltpu
from jax.experimental.pallas import tpu_sc as plsc

N = 50000
E = 1600000
G = 50
B = 128
NUM_ITER = 3

NC = 2
NS = 16
NW = NC * NS
EPW = E // NW
CH = 10000
N_PAD = 51200

TE = 6400
TN = 6400

@functools.lru_cache(maxsize=None)
def _sc_kernels():
    mesh = plsc.VectorSubcoreMesh(
        core_axis_name="c", subcore_axis_name="s",
        num_cores=NC, num_subcores=NS)

    def wid():
        return lax.axis_index("s") * NC + lax.axis_index("c")

    sc_params = pltpu.CompilerParams(
        needs_layout_passes=False, use_tc_tiling_on_sc=False)

    NCH = EPW // CH

    @functools.partial(
        pl.kernel,
        out_type=jax.ShapeDtypeStruct((E,), jnp.float32),
        mesh=mesh,
        scratch_types=[
            pltpu.VMEM((N,), jnp.float32),
            pltpu.VMEM((2, CH), jnp.int32),
            pltpu.VMEM((2, CH), jnp.float32),
            pltpu.SemaphoreType.DMA,
            pltpu.SemaphoreType.DMA,
            pltpu.SemaphoreType.DMA,
            pltpu.SemaphoreType.DMA,
        ],
        compiler_params=sc_params,
    )
    def sc_gather(h_hbm, src_hbm, out_hbm, h_v, idx_v, xj_v,
                  si0, si1, so0, si_h, ):
        w = wid()
        sins = (si0, si1)
        souts = (so0, si_h)
        base0 = w * EPW
        hcp = pltpu.async_copy(h_hbm, h_v, si_h)
        in_cp = [None] * NCH
        out_cp = [None] * NCH
        in_cp[0] = pltpu.async_copy(
            src_hbm.at[pl.ds(base0, CH)], idx_v.at[0], sins[0])
        hcp.wait()
        for ci in range(NCH):
            p = ci % 2
            base = base0 + ci * CH
            in_cp[ci].wait()
            if ci + 1 < NCH:
                in_cp[ci + 1] = pltpu.async_copy(
                    src_hbm.at[pl.ds(base + CH, CH)],
                    idx_v.at[(ci + 1) % 2], sins[(ci + 1) % 2])
            if ci >= 2:
                out_cp[ci - 2].wait()

            def vec_body(j, _):
                idx = idx_v[p, pl.ds(j * 16, 16)]
                xj_v[p, pl.ds(j * 16, 16)] = plsc.load_gather(
                    h_v, [idx])
                return 0

            lax.fori_loop(0, CH // 16, vec_body, 0)
            out_cp[ci] = pltpu.async_copy(
                xj_v.at[p], out_hbm.at[pl.ds(base, CH)], souts[p])
        for ci in range(max(0, NCH - 2), NCH):
            out_cp[ci].wait()

    @functools.partial(
        pl.kernel,
        out_type=jax.ShapeDtypeStruct((NW, N_PAD), jnp.float32),
        mesh=mesh,
        scratch_types=[
            pltpu.VMEM((N_PAD,), jnp.float32),
            pltpu.VMEM((2, CH), jnp.int32),
            pltpu.VMEM((2, CH), jnp.float32),
            pltpu.SemaphoreType.DMA,
            pltpu.SemaphoreType.DMA,
            pltpu.SemaphoreType.DMA,
        ],
        compiler_params=sc_params,
    )
    def sc_scatter_min(msg_hbm, dst_hbm, part_hbm, acc_v, idx_v, msg_v,
                       s0, s1, s_out):
        w = wid()
        sems = (s0, s1)
        base0 = w * EPW
        inf16 = jnp.full((16,), jnp.inf, dtype=jnp.float32)
        in_cp = [None] * NCH
        in_cp[0] = (
            pltpu.async_copy(dst_hbm.at[pl.ds(base0, CH)], idx_v.at[0], s0),
            pltpu.async_copy(msg_hbm.at[pl.ds(base0, CH)], msg_v.at[0], s0),
        )

        def init_body(i, _):
            acc_v[pl.ds(i * 16, 16)] = inf16
            return 0

        lax.fori_loop(0, N_PAD // 16, init_body, 0)

        NPAIR = (CH // 16) // 2
        for ci in range(NCH):
            p = ci % 2
            base = base0 + ci * CH
            in_cp[ci][0].wait()
            in_cp[ci][1].wait()
            if ci + 1 < NCH:
                q = (ci + 1) % 2
                in_cp[ci + 1] = (
                    pltpu.async_copy(
                        dst_hbm.at[pl.ds(base + CH, CH)], idx_v.at[q], sems[q]),
                    pltpu.async_copy(
                        msg_hbm.at[pl.ds(base + CH, CH)], msg_v.at[q], sems[q]),
                )

            def pair_body(j, _):
                o0 = j * 32
                idx0 = idx_v[p, pl.ds(o0, 16)]
                m0 = msg_v[p, pl.ds(o0, 16)]
                idx1 = idx_v[p, pl.ds(o0 + 16, 16)]
                m1 = msg_v[p, pl.ds(o0 + 16, 16)]
                cur0 = plsc.load_gather(acc_v, [idx0])
                cur1 = plsc.load_gather(acc_v, [idx1])
                plsc.store_scatter(acc_v, [idx0], jnp.minimum(cur0, m0))
                plsc.store_scatter(acc_v, [idx1], jnp.minimum(cur1, m1))
                chk0 = plsc.load_gather(acc_v, [idx0])
                chk1 = plsc.load_gather(acc_v, [idx1])

                def fix_body(carry):
                    p0, p1 = carry
                    plsc.store_scatter(acc_v, [idx0], m0, mask=p0)
                    plsc.store_scatter(acc_v, [idx1], m1, mask=p1)
                    c0 = plsc.load_gather(acc_v, [idx0])
                    c1 = plsc.load_gather(acc_v, [idx1])
                    return c0 > m0, c1 > m1

                lax.while_loop(
                    lambda c: jnp.any(c[0]) | jnp.any(c[1]),
                    fix_body, (chk0 > m0, chk1 > m1))
                return 0

            lax.fori_loop(0, NPAIR, pair_body, 0)
        out_cp = pltpu.async_copy(acc_v, part_hbm.at[w], s_out)
        out_cp.wait()

    return sc_gather, sc_scatter_min


def _bdot(a, b):
    return jax.lax.dot(a.astype(jnp.bfloat16), b.astype(jnp.bfloat16),
                       preferred_element_type=jnp.float32)


def _msg_mlp_body(xj_ref, ea_ref, w1t_ref, b1_ref, w2t_ref, b2_ref,
                  w3t_ref, b3_ref, out_ref):
    minp = jnp.concatenate([xj_ref[0], ea_ref[0]], axis=0)
    h1 = jnp.maximum(_bdot(w1t_ref[...], minp) + b1_ref[...], 0.0)
    h2 = jnp.maximum(_bdot(w2t_ref[...], h1) + b2_ref[...], 0.0)
    out_ref[0] = _bdot(w3t_ref[...], h2) + b3_ref[...]


def _tc_msg(xj, ea, w1t, b1c, w2t, b2c, w3t, b3c):
    nblk = E // TE
    xj3 = xj.reshape(nblk, 1, TE)
    ea3 = ea.reshape(nblk, 1, TE)
    full = lambda *s: pl.BlockSpec(s, lambda i: (0,) * len(s))
    out = pl.pallas_call(
        _msg_mlp_body,
        grid=(nblk,),
        in_specs=[
            pl.BlockSpec((1, 1, TE), lambda i: (i, 0, 0)),
            pl.BlockSpec((1, 1, TE), lambda i: (i, 0, 0)),
            full(B, 2), full(B, 1), full(B, B), full(B, 1),
            full(1, B), full(1, 1),
        ],
        out_specs=pl.BlockSpec((1, 1, TE), lambda i: (i, 0, 0)),
        out_shape=jax.ShapeDtypeStruct((nblk, 1, TE), jnp.float32),
    )(xj3, ea3, w1t, b1c, w2t, b2c, w3t, b3c)
    return out.reshape(E)


def _update_body(part_ref, w1c_ref, b1_ref, w2t_ref, b2_ref,
                 w3t_ref, b3_ref, out_ref):
    agg = jnp.min(part_ref[...], axis=0, keepdims=True)
    agg = jnp.where(jnp.isfinite(agg), agg, 0.0)
    h1 = jnp.maximum(_bdot(w1c_ref[...], agg) + b1_ref[...], 0.0)
    h2 = jnp.maximum(_bdot(w2t_ref[...], h1) + b2_ref[...], 0.0)
    out_ref[0] = _bdot(w3t_ref[...], h2) + b3_ref[...]


def _tc_update(part, w1c, b1c, w2t, b2c, w3t, b3c):
    nblk = N_PAD // TN
    full = lambda *s: pl.BlockSpec(s, lambda i: (0,) * len(s))
    out = pl.pallas_call(
        _update_body,
        grid=(nblk,),
        in_specs=[
            pl.BlockSpec((NW, TN), lambda i: (0, i)),
            full(B, 1), full(B, 1), full(B, B), full(B, 1),
            full(1, B), full(1, 1),
        ],
        out_specs=pl.BlockSpec((1, 1, TN), lambda i: (i, 0, 0)),
        out_shape=jax.ShapeDtypeStruct((nblk, 1, TN), jnp.float32),
    )(part, w1c, b1c, w2t, b2c, w3t, b3c)
    return out.reshape(N_PAD)[:N]


def kernel(x, edge_index, edge_attr, mW1, mb1, mW2, mb2, mW3, mb3,
           uW1, ub1, uW2, ub2, uW3, ub3):
    src = edge_index[0]
    dst = edge_index[1]

    mw1t = mW1.T
    mb1c = mb1.reshape(B, 1)
    mw2t = mW2.T
    mb2c = mb2.reshape(B, 1)
    mw3t = mW3.T
    mb3c = mb3.reshape(1, 1)
    uw1c = uW1.reshape(B, 1)
    ub1c = ub1.reshape(B, 1)
    uw2t = uW2.T
    ub2c = ub2.reshape(B, 1)
    uw3t = uW3.T
    ub3c = ub3.reshape(1, 1)

    sc_gather, sc_scatter_min = _sc_kernels()
    h = x.reshape(N)
    for _ in range(NUM_ITER):
        xj = sc_gather(h, src)
        msg = _tc_msg(xj, edge_attr, mw1t, mb1c, mw2t, mb2c, mw3t, mb3c)
        part = sc_scatter_min(msg, dst)
        h = _tc_update(part, uw1c, ub1c, uw2t, ub2c, uw3t, ub3c)
    return h.reshape(G, N // G)
